# baseline (speedup 1.0000x reference)
import functools

import jax
import jax.numpy as jnp
from jax import lax
from jax.experimental import pallas as pl
from jax.experimental.pallas import tpu as pltpu
from jax.experimental.pallas import tpu_sc as plsc

N_NODES = 10000
D_FEAT = 128
N_EDGES = 320000

NC = 2
NS = 16
NW = NC * NS
CHUNK = 128
N_CHUNKS = 80
E_PAD = NW * N_CHUNKS * CHUNK
N_PAD = 10240
ROWS_PER_TILE = N_PAD // NS


def _sc_segment_sum(feature, idx4):
    mesh = plsc.VectorSubcoreMesh(core_axis_name="c", subcore_axis_name="s")

    @functools.partial(
        pl.kernel,
        out_type=jax.ShapeDtypeStruct((NC, N_PAD, D_FEAT), jnp.float32),
        mesh=mesh,
        scratch_types=[
            [pltpu.VMEM((2, CHUNK), jnp.int32)] * 2,
            [pltpu.VMEM((CHUNK, D_FEAT), jnp.float32)] * 2,
            pltpu.VMEM_SHARED((N_PAD, D_FEAT), jnp.float32),
            [pltpu.SemaphoreType.DMA] * 2,
            [pltpu.SemaphoreType.DMA] * 2,
        ],
    )
    def k(feature_hbm, idx_hbm, out_hbm, ibufs, rows, acc_sh, isems, gsems):
        c = lax.axis_index("c")
        s = lax.axis_index("s")
        wid = c * NS + s

        def fire_idx(j, b):
            pltpu.async_copy(idx_hbm.at[wid, j], ibufs[b], isems[b])

        def wait_idx(b):
            pltpu.make_async_copy(idx_hbm.at[wid, 0], ibufs[b],
                                  isems[b]).wait()

        def fire_gather(b):
            pltpu.async_copy(feature_hbm.at[ibufs[b].at[0]], rows[b],
                             gsems[b])

        def wait_gather(b):
            pltpu.make_async_copy(feature_hbm.at[ibufs[b].at[0]], rows[b],
                                  gsems[b]).wait()

        fire_idx(0, 0)
        fire_idx(1, 1)

        zv = jnp.zeros((16,), jnp.float32)

        def zfill(r, _):
            for q in range(D_FEAT // 16):
                rows[0][r, pl.ds(q * 16, 16)] = zv
            return 0

        lax.fori_loop(0, CHUNK, zfill, 0)

        row0 = s * ROWS_PER_TILE
        for rep in range(ROWS_PER_TILE // CHUNK):
            pltpu.sync_copy(rows[0],
                            acc_sh.at[pl.ds(row0 + rep * CHUNK, CHUNK)])

        plsc.subcore_barrier()

        wait_idx(0)
        fire_gather(0)

        def outer(g, _):
            for b in range(2):
                j = g * 2 + b
                wait_gather(b)
                pltpu.sync_copy(rows[b], acc_sh.at[ibufs[b].at[1]],
                                add=True)

                @pl.when(j + 2 < N_CHUNKS)
                def _():
                    fire_idx(j + 2, b)

                @pl.when(j + 1 < N_CHUNKS)
                def _():
                    wait_idx(1 - b)
                    fire_gather(1 - b)
            return 0

        lax.fori_loop(0, N_CHUNKS // 2, outer, 0)

        plsc.subcore_barrier()

        pltpu.sync_copy(
            acc_sh.at[pl.ds(row0, ROWS_PER_TILE)],
            out_hbm.at[c, pl.ds(row0, ROWS_PER_TILE)],
        )

    return k(feature, idx4)


def _tc_apply(p0, p1, feature, W, b2d):
    BR = 2000

    def body(p0_ref, p1_ref, f_ref, w_ref, b_ref, o_ref):
        agg = p0_ref[...] + p1_ref[...]
        z = jnp.dot(agg, w_ref[...], preferred_element_type=jnp.float32)
        o_ref[...] = jnp.maximum(z + b_ref[...], 0.0) + f_ref[...]

    return pl.pallas_call(
        body,
        grid=(N_NODES // BR,),
        in_specs=[
            pl.BlockSpec((BR, D_FEAT), lambda i: (i, 0)),
            pl.BlockSpec((BR, D_FEAT), lambda i: (i, 0)),
            pl.BlockSpec((BR, D_FEAT), lambda i: (i, 0)),
            pl.BlockSpec((D_FEAT, D_FEAT), lambda i: (0, 0)),
            pl.BlockSpec((1, D_FEAT), lambda i: (0, 0)),
        ],
        out_specs=pl.BlockSpec((BR, D_FEAT), lambda i: (i, 0)),
        out_shape=jax.ShapeDtypeStruct((N_NODES, D_FEAT), jnp.float32),
    )(p0, p1, feature, W, b2d)


def kernel(feature, edge_index, W, b):
    npad = E_PAD - N_EDGES
    src = jnp.concatenate(
        [edge_index[0], jnp.zeros((npad,), jnp.int32)])
    dst = jnp.concatenate(
        [edge_index[1], jnp.full((npad,), N_PAD - 1, jnp.int32)])
    idx4 = jnp.stack(
        [src.reshape(NW, N_CHUNKS, CHUNK), dst.reshape(NW, N_CHUNKS, CHUNK)],
        axis=2)
    partials = _sc_segment_sum(feature, idx4)
    return _tc_apply(partials[0, :N_NODES], partials[1, :N_NODES], feature, W,
                     b.reshape(1, D_FEAT))


# speedup vs baseline: 1.0827x; 1.0827x over previous
import functools

import jax
import jax.numpy as jnp
from jax import lax
from jax.experimental import pallas as pl
from jax.experimental.pallas import tpu as pltpu
from jax.experimental.pallas import tpu_sc as plsc

N_NODES = 10000
D_FEAT = 128
N_EDGES = 320000

NC = 2
NS = 16
NW = NC * NS
CHUNK = 128
N_CHUNKS = 80
E_PAD = NW * N_CHUNKS * CHUNK
N_PAD = 10240
ROWS_PER_TILE = N_PAD // NS


def _sc_segment_sum(feature, idx4):
    mesh = plsc.VectorSubcoreMesh(core_axis_name="c", subcore_axis_name="s")

    @functools.partial(
        pl.kernel,
        out_type=jax.ShapeDtypeStruct((NC, N_PAD, D_FEAT), jnp.float32),
        mesh=mesh,
        scratch_types=[
            [pltpu.VMEM((2, CHUNK), jnp.int32)] * 2,
            [pltpu.VMEM((CHUNK, D_FEAT), jnp.float32)] * 2,
            pltpu.VMEM_SHARED((N_PAD, D_FEAT), jnp.float32),
            [pltpu.SemaphoreType.DMA] * 2,
            [pltpu.SemaphoreType.DMA] * 2,
        ],
    )
    def k(feature_hbm, idx_hbm, out_hbm, ibufs, rows, acc_sh, isems, gsems):
        c = lax.axis_index("c")
        s = lax.axis_index("s")
        wid = c * NS + s

        def fire_idx(j, b):
            pltpu.async_copy(idx_hbm.at[wid, j], ibufs[b], isems[b])

        def wait_idx(b):
            pltpu.make_async_copy(idx_hbm.at[wid, 0], ibufs[b],
                                  isems[b]).wait()

        def fire_gather(b):
            pltpu.async_copy(feature_hbm.at[ibufs[b].at[0]], rows[b],
                             gsems[b])

        def wait_gather(b):
            pltpu.make_async_copy(feature_hbm.at[ibufs[b].at[0]], rows[b],
                                  gsems[b]).wait()

        fire_idx(0, 0)
        fire_idx(1, 1)

        zv = jnp.zeros((16,), jnp.float32)

        def zfill(r, _):
            for q in range(D_FEAT // 16):
                rows[0][r, pl.ds(q * 16, 16)] = zv
            return 0

        lax.fori_loop(0, CHUNK, zfill, 0)

        row0 = s * ROWS_PER_TILE
        for rep in range(ROWS_PER_TILE // CHUNK):
            pltpu.sync_copy(rows[0],
                            acc_sh.at[pl.ds(row0 + rep * CHUNK, CHUNK)])

        plsc.subcore_barrier()

        wait_idx(0)
        fire_gather(0)

        def outer(g, _):
            for b in range(2):
                j = g * 2 + b
                wait_gather(b)

                @pl.when(j + 1 < N_CHUNKS)
                def _():
                    wait_idx(1 - b)
                    fire_gather(1 - b)

                pltpu.sync_copy(rows[b], acc_sh.at[ibufs[b].at[1]],
                                add=True)

                @pl.when(j + 2 < N_CHUNKS)
                def _():
                    fire_idx(j + 2, b)
            return 0

        lax.fori_loop(0, N_CHUNKS // 2, outer, 0)

        plsc.subcore_barrier()

        pltpu.sync_copy(
            acc_sh.at[pl.ds(row0, ROWS_PER_TILE)],
            out_hbm.at[c, pl.ds(row0, ROWS_PER_TILE)],
        )

    return k(feature, idx4)


def _tc_apply(p0, p1, feature, W, b2d):
    BR = 2000

    def body(p0_ref, p1_ref, f_ref, w_ref, b_ref, o_ref):
        agg = p0_ref[...] + p1_ref[...]
        z = jnp.dot(agg, w_ref[...], preferred_element_type=jnp.float32)
        o_ref[...] = jnp.maximum(z + b_ref[...], 0.0) + f_ref[...]

    return pl.pallas_call(
        body,
        grid=(N_NODES // BR,),
        in_specs=[
            pl.BlockSpec((BR, D_FEAT), lambda i: (i, 0)),
            pl.BlockSpec((BR, D_FEAT), lambda i: (i, 0)),
            pl.BlockSpec((BR, D_FEAT), lambda i: (i, 0)),
            pl.BlockSpec((D_FEAT, D_FEAT), lambda i: (0, 0)),
            pl.BlockSpec((1, D_FEAT), lambda i: (0, 0)),
        ],
        out_specs=pl.BlockSpec((BR, D_FEAT), lambda i: (i, 0)),
        out_shape=jax.ShapeDtypeStruct((N_NODES, D_FEAT), jnp.float32),
    )(p0, p1, feature, W, b2d)


def kernel(feature, edge_index, W, b):
    npad = E_PAD - N_EDGES
    src = jnp.concatenate(
        [edge_index[0], jnp.zeros((npad,), jnp.int32)])
    dst = jnp.concatenate(
        [edge_index[1], jnp.full((npad,), N_PAD - 1, jnp.int32)])
    idx4 = jnp.stack(
        [src.reshape(NW, N_CHUNKS, CHUNK), dst.reshape(NW, N_CHUNKS, CHUNK)],
        axis=2)
    partials = _sc_segment_sum(feature, idx4)
    return _tc_apply(partials[0, :N_NODES], partials[1, :N_NODES], feature, W,
                     b.reshape(1, D_FEAT))


# speedup vs baseline: 1.9543x; 1.8051x over previous
import functools

import jax
import jax.numpy as jnp
from jax import lax
from jax.experimental import pallas as pl
from jax.experimental.pallas import tpu as pltpu
from jax.experimental.pallas import tpu_sc as plsc

N_NODES = 10000
D_FEAT = 128
N_EDGES = 320000

NC = 2
NS = 16
NW = NC * NS
CHUNK = 80
N_CHUNKS = 126
E_PER_TILE = N_CHUNKS * CHUNK
E_PAD = NW * E_PER_TILE
N_PAD = 10240
ROWS_PER_TILE = N_PAD // NS


def _sc_segment_sum(feature, src, dst):
    mesh = plsc.VectorSubcoreMesh(core_axis_name="c", subcore_axis_name="s")

    @functools.partial(
        pl.kernel,
        out_type=jax.ShapeDtypeStruct((NC, N_PAD, D_FEAT), jnp.float32),
        mesh=mesh,
        scratch_types=[
            [pltpu.VMEM((CHUNK,), jnp.int32)] * 2,
            [pltpu.VMEM((CHUNK,), jnp.int32)] * 2,
            [pltpu.VMEM((CHUNK, D_FEAT), jnp.float32)] * 2,
            pltpu.VMEM_SHARED((N_PAD, D_FEAT), jnp.float32),
            [pltpu.SemaphoreType.DMA] * 2,
            [pltpu.SemaphoreType.DMA] * 2,
        ],
    )
    def k(feature_hbm, src_hbm, dst_hbm, out_hbm, sbufs, dbufs, rows, acc_sh,
          isems, gsems):
        c = lax.axis_index("c")
        s = lax.axis_index("s")
        base = (c * NS + s) * E_PER_TILE

        def fire_idx(j, b):
            off = pl.multiple_of(base + j * CHUNK, 8)
            pltpu.async_copy(src_hbm.at[pl.ds(off, CHUNK)], sbufs[b],
                             isems[b])
            pltpu.async_copy(dst_hbm.at[pl.ds(off, CHUNK)], dbufs[b],
                             isems[b])

        def wait_idx(b):
            pltpu.make_async_copy(src_hbm.at[pl.ds(0, CHUNK)], sbufs[b],
                                  isems[b]).wait()
            pltpu.make_async_copy(dst_hbm.at[pl.ds(0, CHUNK)], dbufs[b],
                                  isems[b]).wait()

        def fire_gather(b):
            pltpu.async_copy(feature_hbm.at[sbufs[b]], rows[b], gsems[b])

        def wait_gather(b):
            pltpu.make_async_copy(feature_hbm.at[sbufs[b]], rows[b],
                                  gsems[b]).wait()

        fire_idx(0, 0)
        fire_idx(1, 1)

        zv = jnp.zeros((16,), jnp.float32)

        def zfill(r, _):
            for q in range(D_FEAT // 16):
                rows[0][r, pl.ds(q * 16, 16)] = zv
            return 0

        lax.fori_loop(0, CHUNK, zfill, 0)

        row0 = s * ROWS_PER_TILE
        for rep in range(ROWS_PER_TILE // CHUNK):
            pltpu.sync_copy(rows[0],
                            acc_sh.at[pl.ds(row0 + rep * CHUNK, CHUNK)])

        plsc.subcore_barrier()

        wait_idx(0)
        fire_gather(0)

        def outer(g, _):
            for b in range(2):
                j = g * 2 + b
                wait_gather(b)

                @pl.when(j + 1 < N_CHUNKS)
                def _():
                    wait_idx(1 - b)
                    fire_gather(1 - b)

                pltpu.sync_copy(rows[b], acc_sh.at[dbufs[b]], add=True)

                @pl.when(j + 2 < N_CHUNKS)
                def _():
                    fire_idx(j + 2, b)
            return 0

        lax.fori_loop(0, N_CHUNKS // 2, outer, 0)

        plsc.subcore_barrier()

        pltpu.sync_copy(
            acc_sh.at[pl.ds(row0, ROWS_PER_TILE)],
            out_hbm.at[c, pl.ds(row0, ROWS_PER_TILE)],
        )

    return k(feature, src, dst)


def _tc_apply(p0, p1, feature, W, b2d):
    BR = 2000

    def body(p0_ref, p1_ref, f_ref, w_ref, b_ref, o_ref):
        agg = p0_ref[...] + p1_ref[...]
        z = jnp.dot(agg, w_ref[...], preferred_element_type=jnp.float32)
        o_ref[...] = jnp.maximum(z + b_ref[...], 0.0) + f_ref[...]

    return pl.pallas_call(
        body,
        grid=(N_NODES // BR,),
        in_specs=[
            pl.BlockSpec((BR, D_FEAT), lambda i: (i, 0)),
            pl.BlockSpec((BR, D_FEAT), lambda i: (i, 0)),
            pl.BlockSpec((BR, D_FEAT), lambda i: (i, 0)),
            pl.BlockSpec((D_FEAT, D_FEAT), lambda i: (0, 0)),
            pl.BlockSpec((1, D_FEAT), lambda i: (0, 0)),
        ],
        out_specs=pl.BlockSpec((BR, D_FEAT), lambda i: (i, 0)),
        out_shape=jax.ShapeDtypeStruct((N_NODES, D_FEAT), jnp.float32),
    )(p0, p1, feature, W, b2d)


def kernel(feature, edge_index, W, b):
    npad = E_PAD - N_EDGES
    src = jnp.concatenate(
        [edge_index[0], jnp.zeros((npad,), jnp.int32)])
    dst = jnp.concatenate(
        [edge_index[1], jnp.full((npad,), N_PAD - 1, jnp.int32)])
    partials = _sc_segment_sum(feature, src, dst)
    return _tc_apply(partials[0, :N_NODES], partials[1, :N_NODES], feature, W,
                     b.reshape(1, D_FEAT))
